# Initial kernel scaffold; baseline (speedup 1.0000x reference)
import functools

import jax
import jax.numpy as jnp
from jax import lax
from jax.experimental import pallas as pl
from jax.experimental.pallas import tpu as pltpu
from jax.experimental.pallas import tpu_sc as plsc

_N = 10000
_NP = 10240
_E = 320000
_EPAD = 327680
_NC, _NS = 2, 16
_NW = _NC * _NS
_CH = 128
_NCH = _EPAD // (_NW * _CH)
_RPT = _NP // _NS

_f32 = jnp.float32


@functools.cache
def _sc_scatter(w, with_gather):
    mesh = plsc.VectorSubcoreMesh(core_axis_name="c", subcore_axis_name="s")
    scratch = [
        pltpu.VMEM((_NCH, _CH), jnp.int32),
        pltpu.VMEM((_CH, w), _f32),
        pltpu.VMEM_SHARED((_NP, w), _f32),
    ]
    if with_gather:
        scratch.insert(0, pltpu.VMEM((_NCH, _CH), jnp.int32))

    def body(*refs):
        if with_gather:
            (u_hbm, z_hbm, row_hbm, col_hbm, out_hbm,
             rowi, coli, gbuf, acc) = refs
        else:
            (ones_hbm, z_hbm, col_hbm, out_hbm, coli, gbuf, acc) = refs
        cid = lax.axis_index("c")
        sid = lax.axis_index("s")
        wid = cid * _NS + sid
        pltpu.sync_copy(col_hbm.at[wid], coli)
        if with_gather:
            pltpu.sync_copy(row_hbm.at[wid], rowi)
        else:
            pltpu.sync_copy(ones_hbm, gbuf)
        pltpu.sync_copy(z_hbm.at[pl.ds(sid * _RPT, _RPT)],
                        acc.at[pl.ds(sid * _RPT, _RPT)])
        plsc.subcore_barrier()

        def chunk(j, carry):
            if with_gather:
                pltpu.sync_copy(u_hbm.at[rowi.at[j]], gbuf)
            pltpu.sync_copy(gbuf, acc.at[coli.at[j]], add=True)
            return carry

        lax.fori_loop(0, _NCH, chunk, 0, unroll=False)
        plsc.subcore_barrier()
        pltpu.sync_copy(acc.at[pl.ds(sid * _RPT, _RPT)],
                        out_hbm.at[cid, pl.ds(sid * _RPT, _RPT)])

    return pl.kernel(
        body,
        out_type=jax.ShapeDtypeStruct((_NC, _NP, w), _f32),
        mesh=mesh,
        scratch_types=scratch,
    )


def _tc(body, out_shapes, name):
    return pl.pallas_call(body, out_shape=out_shapes, name=name)


def _selfidx(rowf, colf):
    def body(r_ref, c_ref, o_ref):
        r = r_ref[...]
        o_ref[...] = jnp.where(r == c_ref[...], r, _N)
    return _tc(body, jax.ShapeDtypeStruct(rowf.shape, jnp.int32),
               "selfidx")(rowf, colf)


def _deg_b1(dparts, sparts, x, wc, do):
    def body(dp, sp, x_ref, wc_ref, dis_ref, coef_ref, p_ref, u_ref):
        dincl = dp[0, :, 0:1] + dp[1, :, 0:1]
        scnt = sp[0, :, 0:1] + sp[1, :, 0:1]
        deg = dincl - scnt
        pos = deg > 0
        dis = jnp.where(pos, 1.0 / jnp.sqrt(jnp.maximum(deg, 1e-12)), 0.0)
        dis_ref[...] = dis
        coef_ref[...] = jnp.where(pos, scnt * dis * dis, -1.0)
        p = jnp.dot(x_ref[...], wc_ref[...], preferred_element_type=_f32)
        p_ref[...] = p
        u_ref[...] = dis * p[:, 2 * do:3 * do]
    outs = (jax.ShapeDtypeStruct((_NP, 1), _f32),
            jax.ShapeDtypeStruct((_NP, 1), _f32),
            jax.ShapeDtypeStruct((_NP, 3 * do), _f32),
            jax.ShapeDtypeStruct((_NP, do), _f32))
    return _tc(body, outs, "deg_b1")(dparts, sparts, x, wc)


def _b2(g, p, dis, coef, do):
    def body(g_ref, p_ref, dis_ref, coef_ref, a_ref, ua_ref):
        dis_v = dis_ref[...]
        gsum = g_ref[0] + g_ref[1]
        p2 = p_ref[:, 2 * do:3 * do]
        t = coef_ref[...] * p2 - dis_v * gsum
        a = p_ref[:, do:2 * do] + 2.0 * t
        a_ref[...] = a
        ua_ref[...] = dis_v * a
    outs = (jax.ShapeDtypeStruct((_NP, do), _f32),
            jax.ShapeDtypeStruct((_NP, do), _f32))
    return _tc(body, outs, "b2")(g, p, dis, coef)


def _b3_h(g_ref, a_ref, p_ref, b_ref, dis_ref, coef_ref, do):
    gsum = g_ref[0] + g_ref[1]
    y = coef_ref[...] * a_ref[...] - dis_ref[...] * gsum
    h = p_ref[:, 0:do] - p_ref[:, 2 * do:3 * do] + y + b_ref[...]
    return jnp.maximum(h, 0.0)


def _b3_b1(g, a, p, b, dis, coef, wc_next, do, do_next):
    def body(g_ref, a_ref, p_ref, b_ref, dis_ref, coef_ref, wc_ref,
             pn_ref, un_ref):
        h = _b3_h(g_ref, a_ref, p_ref, b_ref, dis_ref, coef_ref, do)
        pn = jnp.dot(h, wc_ref[...], preferred_element_type=_f32)
        pn_ref[...] = pn
        un_ref[...] = dis_ref[...] * pn[:, 2 * do_next:3 * do_next]
    outs = (jax.ShapeDtypeStruct((_NP, 3 * do_next), _f32),
            jax.ShapeDtypeStruct((_NP, do_next), _f32))
    return _tc(body, outs, "b3_b1")(g, a, p, b, dis, coef, wc_next)


def _b3_a1(g, a, p, b, dis, coef, do):
    def body(g_ref, a_ref, p_ref, b_ref, dis_ref, coef_ref, h_ref, u_ref):
        h = _b3_h(g_ref, a_ref, p_ref, b_ref, dis_ref, coef_ref, do)
        h_ref[...] = h
        u_ref[...] = dis_ref[...] * h
    outs = (jax.ShapeDtypeStruct((_NP, do), _f32),
            jax.ShapeDtypeStruct((_NP, do), _f32))
    return _tc(body, outs, "b3_a1")(g, a, p, b, dis, coef)


def _a2(g, x, dis, coef, di):
    def body(g_ref, x_ref, dis_ref, coef_ref, t1_ref, u1_ref):
        dis_v = dis_ref[...]
        gsum = g_ref[0] + g_ref[1]
        t1 = coef_ref[...] * x_ref[...] - dis_v * gsum
        t1_ref[...] = t1
        u1_ref[...] = dis_v * t1
    outs = (jax.ShapeDtypeStruct((_NP, di), _f32),
            jax.ShapeDtypeStruct((_NP, di), _f32))
    return _tc(body, outs, "a2")(g, x, dis, coef)


def _a3_h(g1_ref, t1_ref, x_ref, w_ref, b_ref, dis_ref, coef_ref):
    g1 = g1_ref[0] + g1_ref[1]
    t1 = t1_ref[...]
    xv = x_ref[...]
    t2 = 2.0 * (coef_ref[...] * t1 - dis_ref[...] * g1) - xv
    h = (jnp.dot(xv, w_ref[0], preferred_element_type=_f32)
         + jnp.dot(t1, w_ref[1], preferred_element_type=_f32)
         + jnp.dot(t2, w_ref[2], preferred_element_type=_f32)
         + b_ref[...])
    return jnp.maximum(h, 0.0)


def _a3_b1(g1, t1, x, w, b, dis, coef, wc_next, do, do_next):
    def body(g1_ref, t1_ref, x_ref, w_ref, b_ref, dis_ref, coef_ref,
             wc_ref, pn_ref, un_ref):
        h = _a3_h(g1_ref, t1_ref, x_ref, w_ref, b_ref, dis_ref, coef_ref)
        pn = jnp.dot(h, wc_ref[...], preferred_element_type=_f32)
        pn_ref[...] = pn
        un_ref[...] = dis_ref[...] * pn[:, 2 * do_next:3 * do_next]
    outs = (jax.ShapeDtypeStruct((_NP, 3 * do_next), _f32),
            jax.ShapeDtypeStruct((_NP, do_next), _f32))
    return _tc(body, outs, "a3_b1")(g1, t1, x, w, b, dis, coef, wc_next)


def _a3_final(g1, t1, x, w, b, dis, coef, lwt, lb, bat):
    def body(g1_ref, t1_ref, x_ref, w_ref, b_ref, dis_ref, coef_ref,
             lwt_ref, lb_ref, bat_ref, out_ref):
        h = _a3_h(g1_ref, t1_ref, x_ref, w_ref, b_ref, dis_ref, coef_ref)
        seg = lax.broadcasted_iota(jnp.int32, (16, _NP), 0)
        oh = (seg == bat_ref[...]).astype(_f32)
        sums = jnp.dot(oh, h, preferred_element_type=_f32)
        cnts = jnp.sum(oh, axis=1, keepdims=True)
        pooled = sums / jnp.maximum(cnts, 1.0)
        logits = jnp.dot(pooled, lwt_ref[...],
                         preferred_element_type=_f32) + lb_ref[...]
        m = jnp.max(logits, axis=1, keepdims=True)
        e = jnp.exp(logits - m)
        out_ref[...] = logits - m - jnp.log(jnp.sum(e, axis=1, keepdims=True))
    return _tc(body, jax.ShapeDtypeStruct((16, 10), _f32),
               "a3_final")(g1, t1, x, w, b, dis, coef, lwt, lb, bat)


def kernel(x, W1, b1, W2, b2, W3, b3, W4, b4, W5, b5, linW, linb,
           edge_index, batch):
    x = x.astype(_f32)
    row = edge_index[0].astype(jnp.int32)
    col = edge_index[1].astype(jnp.int32)
    npad = _EPAD - _E
    fill = jnp.full((npad,), _N, jnp.int32)
    rowf = jnp.concatenate([row, fill]).reshape(_EPAD // _CH, _CH)
    colf = jnp.concatenate([col, fill]).reshape(_EPAD // _CH, _CH)
    selff = _selfidx(rowf, colf)
    rowp = rowf.reshape(_NW, _NCH, _CH)
    colp = colf.reshape(_NW, _NCH, _CH)
    selfp = selff.reshape(_NW, _NCH, _CH)

    ones_ch = jnp.ones((_CH, 16), _f32)
    zeros = {wi: jnp.zeros((_NP, wi), _f32) for wi in (16, 32, 64)}

    dparts = _sc_scatter(16, False)(ones_ch, zeros[16], rowp)
    sparts = _sc_scatter(16, False)(ones_ch, zeros[16], selfp)

    def g_of(u, wi):
        return _sc_scatter(wi, True)(u, zeros[wi], rowp, colp)

    xp = jnp.pad(x, ((0, _NP - _N), (0, 0)))
    wc1 = jnp.concatenate([W1[0], W1[1], W1[2]], axis=1)
    wc2 = jnp.concatenate([W2[0], W2[1], W2[2]], axis=1)
    wc4 = jnp.concatenate([W4[0], W4[1], W4[2]], axis=1)
    b1r, b2r, b4r = b1.reshape(1, -1), b2.reshape(1, -1), b4.reshape(1, -1)

    dis, coef, p, u = _deg_b1(dparts, sparts, xp, wc1, 32)
    a, ua = _b2(g_of(u, 32), p, dis, coef, 32)
    p, u = _b3_b1(g_of(ua, 32), a, p, b1r, dis, coef, wc2, 32, 32)
    a, ua = _b2(g_of(u, 32), p, dis, coef, 32)
    h, u = _b3_a1(g_of(ua, 32), a, p, b2r, dis, coef, 32)
    t1, u1 = _a2(g_of(u, 32), h, dis, coef, 32)
    p, u = _a3_b1(g_of(u1, 32), t1, h, W3, b3.reshape(1, -1), dis, coef,
                  wc4, 64, 64)
    a, ua = _b2(g_of(u, 64), p, dis, coef, 64)
    h, u = _b3_a1(g_of(ua, 64), a, p, b4r, dis, coef, 64)
    t1, u1 = _a2(g_of(u, 64), h, dis, coef, 64)
    batp = jnp.concatenate(
        [batch.astype(jnp.int32), jnp.full((_NP - _N,), 16, jnp.int32)]
    ).reshape(1, _NP)
    return _a3_final(g_of(u1, 64), t1, h, W5, b5.reshape(1, -1), dis, coef,
                     linW.T, linb.reshape(1, -1), batp)


# speedup vs baseline: 6.8445x; 6.8445x over previous
import functools

import jax
import jax.numpy as jnp
from jax import lax
from jax.experimental import pallas as pl
from jax.experimental.pallas import tpu as pltpu
from jax.experimental.pallas import tpu_sc as plsc

_N = 10000
_NP = 10240
_E = 320000
_EPAD = 327680
_NC, _NS = 2, 16
_NW = _NC * _NS
_CH = 128
_NCH = _EPAD // (_NW * _CH)
_RPT = _NP // _NS

_f32 = jnp.float32


@functools.cache
def _sc_scatter(w, with_gather):
    mesh = plsc.VectorSubcoreMesh(core_axis_name="c", subcore_axis_name="s")
    scratch = [
        pltpu.VMEM((_NCH, _CH), jnp.int32),
        pltpu.VMEM((_CH, w), _f32),
        pltpu.VMEM_SHARED((_NP, w), _f32),
    ]
    if with_gather:
        scratch.insert(0, pltpu.VMEM((_NCH, _CH), jnp.int32))

    def body(*refs):
        if with_gather:
            (u_hbm, z_hbm, row_hbm, col_hbm, out_hbm,
             rowi, coli, gbuf, acc) = refs
        else:
            (ones_hbm, z_hbm, col_hbm, out_hbm, coli, gbuf, acc) = refs
        cid = lax.axis_index("c")
        sid = lax.axis_index("s")
        wid = cid * _NS + sid
        pltpu.sync_copy(col_hbm.at[wid], coli)
        if with_gather:
            pltpu.sync_copy(row_hbm.at[wid], rowi)
        else:
            pltpu.sync_copy(ones_hbm, gbuf)
        pltpu.sync_copy(z_hbm.at[pl.ds(sid * _RPT, _RPT)],
                        acc.at[pl.ds(sid * _RPT, _RPT)])
        plsc.subcore_barrier()

        def chunk(j, carry):
            if with_gather:
                pltpu.sync_copy(u_hbm.at[rowi.at[j]], gbuf)
            pltpu.sync_copy(gbuf, acc.at[coli.at[j]], add=True)
            return carry

        lax.fori_loop(0, _NCH, chunk, 0, unroll=False)
        plsc.subcore_barrier()
        pltpu.sync_copy(acc.at[pl.ds(sid * _RPT, _RPT)],
                        out_hbm.at[cid, pl.ds(sid * _RPT, _RPT)])

    return pl.kernel(
        body,
        out_type=jax.ShapeDtypeStruct((_NC, _NP, w), _f32),
        mesh=mesh,
        scratch_types=scratch,
        compiler_params=pltpu.CompilerParams(use_tc_tiling_on_sc=False),
    )


def _tc(body, out_shapes, name):
    return pl.pallas_call(body, out_shape=out_shapes, name=name)


def _selfidx(rowf, colf):
    def body(r_ref, c_ref, o_ref):
        r = r_ref[...]
        o_ref[...] = jnp.where(r == c_ref[...], r, _N)
    return _tc(body, jax.ShapeDtypeStruct(rowf.shape, jnp.int32),
               "selfidx")(rowf, colf)


def _deg_b1(dparts, sparts, x, wc, do):
    def body(dp, sp, x_ref, wc_ref, dis_ref, coef_ref, p_ref, u_ref):
        dincl = dp[0, :, 0:1] + dp[1, :, 0:1]
        scnt = sp[0, :, 0:1] + sp[1, :, 0:1]
        deg = dincl - scnt
        pos = deg > 0
        dis = jnp.where(pos, 1.0 / jnp.sqrt(jnp.maximum(deg, 1e-12)), 0.0)
        dis_ref[...] = dis
        coef_ref[...] = jnp.where(pos, scnt * dis * dis, -1.0)
        p = jnp.dot(x_ref[...], wc_ref[...], preferred_element_type=_f32)
        p_ref[...] = p
        u_ref[...] = dis * p[:, 2 * do:3 * do]
    outs = (jax.ShapeDtypeStruct((_NP, 1), _f32),
            jax.ShapeDtypeStruct((_NP, 1), _f32),
            jax.ShapeDtypeStruct((_NP, 3 * do), _f32),
            jax.ShapeDtypeStruct((_NP, do), _f32))
    return _tc(body, outs, "deg_b1")(dparts, sparts, x, wc)


def _b2(g, p, dis, coef, do):
    def body(g_ref, p_ref, dis_ref, coef_ref, a_ref, ua_ref):
        dis_v = dis_ref[...]
        gsum = g_ref[0] + g_ref[1]
        p2 = p_ref[:, 2 * do:3 * do]
        t = coef_ref[...] * p2 - dis_v * gsum
        a = p_ref[:, do:2 * do] + 2.0 * t
        a_ref[...] = a
        ua_ref[...] = dis_v * a
    outs = (jax.ShapeDtypeStruct((_NP, do), _f32),
            jax.ShapeDtypeStruct((_NP, do), _f32))
    return _tc(body, outs, "b2")(g, p, dis, coef)


def _b3_h(g_ref, a_ref, p_ref, b_ref, dis_ref, coef_ref, do):
    gsum = g_ref[0] + g_ref[1]
    y = coef_ref[...] * a_ref[...] - dis_ref[...] * gsum
    h = p_ref[:, 0:do] - p_ref[:, 2 * do:3 * do] + y + b_ref[...]
    return jnp.maximum(h, 0.0)


def _b3_b1(g, a, p, b, dis, coef, wc_next, do, do_next):
    def body(g_ref, a_ref, p_ref, b_ref, dis_ref, coef_ref, wc_ref,
             pn_ref, un_ref):
        h = _b3_h(g_ref, a_ref, p_ref, b_ref, dis_ref, coef_ref, do)
        pn = jnp.dot(h, wc_ref[...], preferred_element_type=_f32)
        pn_ref[...] = pn
        un_ref[...] = dis_ref[...] * pn[:, 2 * do_next:3 * do_next]
    outs = (jax.ShapeDtypeStruct((_NP, 3 * do_next), _f32),
            jax.ShapeDtypeStruct((_NP, do_next), _f32))
    return _tc(body, outs, "b3_b1")(g, a, p, b, dis, coef, wc_next)


def _b3_a1(g, a, p, b, dis, coef, do):
    def body(g_ref, a_ref, p_ref, b_ref, dis_ref, coef_ref, h_ref, u_ref):
        h = _b3_h(g_ref, a_ref, p_ref, b_ref, dis_ref, coef_ref, do)
        h_ref[...] = h
        u_ref[...] = dis_ref[...] * h
    outs = (jax.ShapeDtypeStruct((_NP, do), _f32),
            jax.ShapeDtypeStruct((_NP, do), _f32))
    return _tc(body, outs, "b3_a1")(g, a, p, b, dis, coef)


def _a2(g, x, dis, coef, di):
    def body(g_ref, x_ref, dis_ref, coef_ref, t1_ref, u1_ref):
        dis_v = dis_ref[...]
        gsum = g_ref[0] + g_ref[1]
        t1 = coef_ref[...] * x_ref[...] - dis_v * gsum
        t1_ref[...] = t1
        u1_ref[...] = dis_v * t1
    outs = (jax.ShapeDtypeStruct((_NP, di), _f32),
            jax.ShapeDtypeStruct((_NP, di), _f32))
    return _tc(body, outs, "a2")(g, x, dis, coef)


def _a3_h(g1_ref, t1_ref, x_ref, w_ref, b_ref, dis_ref, coef_ref):
    g1 = g1_ref[0] + g1_ref[1]
    t1 = t1_ref[...]
    xv = x_ref[...]
    t2 = 2.0 * (coef_ref[...] * t1 - dis_ref[...] * g1) - xv
    h = (jnp.dot(xv, w_ref[0], preferred_element_type=_f32)
         + jnp.dot(t1, w_ref[1], preferred_element_type=_f32)
         + jnp.dot(t2, w_ref[2], preferred_element_type=_f32)
         + b_ref[...])
    return jnp.maximum(h, 0.0)


def _a3_b1(g1, t1, x, w, b, dis, coef, wc_next, do, do_next):
    def body(g1_ref, t1_ref, x_ref, w_ref, b_ref, dis_ref, coef_ref,
             wc_ref, pn_ref, un_ref):
        h = _a3_h(g1_ref, t1_ref, x_ref, w_ref, b_ref, dis_ref, coef_ref)
        pn = jnp.dot(h, wc_ref[...], preferred_element_type=_f32)
        pn_ref[...] = pn
        un_ref[...] = dis_ref[...] * pn[:, 2 * do_next:3 * do_next]
    outs = (jax.ShapeDtypeStruct((_NP, 3 * do_next), _f32),
            jax.ShapeDtypeStruct((_NP, do_next), _f32))
    return _tc(body, outs, "a3_b1")(g1, t1, x, w, b, dis, coef, wc_next)


def _a3_final(g1, t1, x, w, b, dis, coef, lwt, lb, bat):
    def body(g1_ref, t1_ref, x_ref, w_ref, b_ref, dis_ref, coef_ref,
             lwt_ref, lb_ref, bat_ref, out_ref):
        h = _a3_h(g1_ref, t1_ref, x_ref, w_ref, b_ref, dis_ref, coef_ref)
        seg = lax.broadcasted_iota(jnp.int32, (16, _NP), 0)
        oh = (seg == bat_ref[...]).astype(_f32)
        sums = jnp.dot(oh, h, preferred_element_type=_f32)
        cnts = jnp.sum(oh, axis=1, keepdims=True)
        pooled = sums / jnp.maximum(cnts, 1.0)
        logits = jnp.dot(pooled, lwt_ref[...],
                         preferred_element_type=_f32) + lb_ref[...]
        m = jnp.max(logits, axis=1, keepdims=True)
        e = jnp.exp(logits - m)
        out_ref[...] = logits - m - jnp.log(jnp.sum(e, axis=1, keepdims=True))
    return _tc(body, jax.ShapeDtypeStruct((16, 10), _f32),
               "a3_final")(g1, t1, x, w, b, dis, coef, lwt, lb, bat)


def kernel(x, W1, b1, W2, b2, W3, b3, W4, b4, W5, b5, linW, linb,
           edge_index, batch):
    x = x.astype(_f32)
    row = edge_index[0].astype(jnp.int32)
    col = edge_index[1].astype(jnp.int32)
    npad = _EPAD - _E
    fill = jnp.full((npad,), _N, jnp.int32)
    rowf = jnp.concatenate([row, fill]).reshape(_EPAD // _CH, _CH)
    colf = jnp.concatenate([col, fill]).reshape(_EPAD // _CH, _CH)
    selff = _selfidx(rowf, colf)
    rowp = rowf.reshape(_NW, _NCH, _CH)
    colp = colf.reshape(_NW, _NCH, _CH)
    selfp = selff.reshape(_NW, _NCH, _CH)

    ones_ch = jnp.ones((_CH, 16), _f32)
    zeros = {wi: jnp.zeros((_NP, wi), _f32) for wi in (16, 32, 64)}

    dparts = _sc_scatter(16, False)(ones_ch, zeros[16], rowp)
    sparts = _sc_scatter(16, False)(ones_ch, zeros[16], selfp)

    def g_of(u, wi):
        return _sc_scatter(wi, True)(u, zeros[wi], rowp, colp)

    xp = jnp.pad(x, ((0, _NP - _N), (0, 0)))
    wc1 = jnp.concatenate([W1[0], W1[1], W1[2]], axis=1)
    wc2 = jnp.concatenate([W2[0], W2[1], W2[2]], axis=1)
    wc4 = jnp.concatenate([W4[0], W4[1], W4[2]], axis=1)
    b1r, b2r, b4r = b1.reshape(1, -1), b2.reshape(1, -1), b4.reshape(1, -1)

    dis, coef, p, u = _deg_b1(dparts, sparts, xp, wc1, 32)
    a, ua = _b2(g_of(u, 32), p, dis, coef, 32)
    p, u = _b3_b1(g_of(ua, 32), a, p, b1r, dis, coef, wc2, 32, 32)
    a, ua = _b2(g_of(u, 32), p, dis, coef, 32)
    h, u = _b3_a1(g_of(ua, 32), a, p, b2r, dis, coef, 32)
    t1, u1 = _a2(g_of(u, 32), h, dis, coef, 32)
    p, u = _a3_b1(g_of(u1, 32), t1, h, W3, b3.reshape(1, -1), dis, coef,
                  wc4, 64, 64)
    a, ua = _b2(g_of(u, 64), p, dis, coef, 64)
    h, u = _b3_a1(g_of(ua, 64), a, p, b4r, dis, coef, 64)
    t1, u1 = _a2(g_of(u, 64), h, dis, coef, 64)
    batp = jnp.concatenate(
        [batch.astype(jnp.int32), jnp.full((_NP - _N,), 16, jnp.int32)]
    ).reshape(1, _NP)
    return _a3_final(g_of(u1, 64), t1, h, W5, b5.reshape(1, -1), dis, coef,
                     linW.T, linb.reshape(1, -1), batp)


# speedup vs baseline: 8.5048x; 1.2426x over previous
import functools

import jax
import jax.numpy as jnp
from jax import lax
from jax.experimental import pallas as pl
from jax.experimental.pallas import tpu as pltpu
from jax.experimental.pallas import tpu_sc as plsc

_N = 10000
_NP = 10240
_E = 320000
_EPAD = 327680
_NC, _NS = 2, 16
_NW = _NC * _NS
_CH = 128
_NCH = _EPAD // (_NW * _CH)
_RPT = _NP // _NS

_f32 = jnp.float32


@functools.cache
def _sc_scatter(w, with_gather):
    mesh = plsc.VectorSubcoreMesh(core_axis_name="c", subcore_axis_name="s")
    nbuf = 4
    scratch = [
        pltpu.VMEM((_NCH, _CH), jnp.int32),
        pltpu.VMEM((nbuf if with_gather else 1, _CH, w), _f32),
        pltpu.VMEM_SHARED((_NP, w), _f32),
    ] + [pltpu.SemaphoreType.DMA] * nbuf
    if with_gather:
        scratch = ([pltpu.VMEM((_NCH, _CH), jnp.int32)] + scratch
                   + [pltpu.SemaphoreType.DMA] * nbuf)

    def body(*refs):
        if with_gather:
            (u_hbm, z_hbm, row_hbm, col_hbm, out_hbm,
             rowi, coli, gbuf, acc, *sems) = refs
            ssem, gsem = sems[:nbuf], sems[nbuf:]
        else:
            (ones_hbm, z_hbm, col_hbm, out_hbm, coli, gbuf, acc,
             *ssem) = refs
        cid = lax.axis_index("c")
        sid = lax.axis_index("s")
        wid = cid * _NS + sid
        pltpu.sync_copy(col_hbm.at[wid], coli)
        if with_gather:
            pltpu.sync_copy(row_hbm.at[wid], rowi)
        else:
            pltpu.sync_copy(ones_hbm, gbuf.at[0])
        pltpu.sync_copy(z_hbm.at[pl.ds(sid * _RPT, _RPT)],
                        acc.at[pl.ds(sid * _RPT, _RPT)])
        plsc.subcore_barrier()

        def gstart(j, b):
            pltpu.async_copy(u_hbm.at[rowi.at[j]], gbuf.at[b], gsem[b])

        def gwait(j, b):
            pltpu.make_async_copy(u_hbm.at[rowi.at[j]], gbuf.at[b],
                                  gsem[b]).wait()

        def sstart(j, b):
            pltpu.async_copy(gbuf.at[b if with_gather else 0],
                             acc.at[coli.at[j]], ssem[b], add=True)

        def swait(j, b):
            pltpu.make_async_copy(gbuf.at[b if with_gather else 0],
                                  acc.at[coli.at[j]], ssem[b]).wait()

        if with_gather:
            gstart(0, 0)
            gstart(1, 1)
            gstart(2, 2)
            gwait(0, 0)
            sstart(0, 0)
            gstart(3, 3)
            gwait(1, 1)
            sstart(1, 1)

            @pl.loop(2, _NCH - 2, step=nbuf)
            def _(g):
                for b in range(nbuf):
                    j = g + b
                    bcur = (2 + b) % nbuf
                    bplus = (2 + b + 2) % nbuf
                    swait(j - 2, bplus)
                    gstart(j + 2, bplus)
                    gwait(j, bcur)
                    sstart(j, bcur)

            gwait(_NCH - 2, (_NCH - 2) % nbuf)
            sstart(_NCH - 2, (_NCH - 2) % nbuf)
            gwait(_NCH - 1, (_NCH - 1) % nbuf)
            sstart(_NCH - 1, (_NCH - 1) % nbuf)
            for k in range(nbuf):
                j = _NCH - nbuf + k
                swait(j, j % nbuf)
        else:
            for j in range(nbuf):
                sstart(j, j)

            @pl.loop(nbuf, _NCH, step=nbuf)
            def _(g):
                for b in range(nbuf):
                    j = g + b
                    swait(j - nbuf, b)
                    sstart(j, b)

            for k in range(nbuf):
                j = _NCH - nbuf + k
                swait(j, k)
        plsc.subcore_barrier()
        pltpu.sync_copy(acc.at[pl.ds(sid * _RPT, _RPT)],
                        out_hbm.at[cid, pl.ds(sid * _RPT, _RPT)])

    return pl.kernel(
        body,
        out_type=jax.ShapeDtypeStruct((_NC, _NP, w), _f32),
        mesh=mesh,
        scratch_types=scratch,
        compiler_params=pltpu.CompilerParams(use_tc_tiling_on_sc=False),
    )


def _tc(body, out_shapes, name):
    return pl.pallas_call(body, out_shape=out_shapes, name=name)


def _selfidx(rowf, colf):
    def body(r_ref, c_ref, o_ref):
        r = r_ref[...]
        o_ref[...] = jnp.where(r == c_ref[...], r, _N)
    return _tc(body, jax.ShapeDtypeStruct(rowf.shape, jnp.int32),
               "selfidx")(rowf, colf)


def _deg_b1(dparts, sparts, x, wc, do):
    def body(dp, sp, x_ref, wc_ref, dis_ref, coef_ref, p_ref, u_ref):
        dincl = dp[0, :, 0:1] + dp[1, :, 0:1]
        scnt = sp[0, :, 0:1] + sp[1, :, 0:1]
        deg = dincl - scnt
        pos = deg > 0
        dis = jnp.where(pos, 1.0 / jnp.sqrt(jnp.maximum(deg, 1e-12)), 0.0)
        dis_ref[...] = dis
        coef_ref[...] = jnp.where(pos, scnt * dis * dis, -1.0)
        p = jnp.dot(x_ref[...], wc_ref[...], preferred_element_type=_f32)
        p_ref[...] = p
        u_ref[...] = dis * p[:, 2 * do:3 * do]
    outs = (jax.ShapeDtypeStruct((_NP, 1), _f32),
            jax.ShapeDtypeStruct((_NP, 1), _f32),
            jax.ShapeDtypeStruct((_NP, 3 * do), _f32),
            jax.ShapeDtypeStruct((_NP, do), _f32))
    return _tc(body, outs, "deg_b1")(dparts, sparts, x, wc)


def _b2(g, p, dis, coef, do):
    def body(g_ref, p_ref, dis_ref, coef_ref, a_ref, ua_ref):
        dis_v = dis_ref[...]
        gsum = g_ref[0] + g_ref[1]
        p2 = p_ref[:, 2 * do:3 * do]
        t = coef_ref[...] * p2 - dis_v * gsum
        a = p_ref[:, do:2 * do] + 2.0 * t
        a_ref[...] = a
        ua_ref[...] = dis_v * a
    outs = (jax.ShapeDtypeStruct((_NP, do), _f32),
            jax.ShapeDtypeStruct((_NP, do), _f32))
    return _tc(body, outs, "b2")(g, p, dis, coef)


def _b3_h(g_ref, a_ref, p_ref, b_ref, dis_ref, coef_ref, do):
    gsum = g_ref[0] + g_ref[1]
    y = coef_ref[...] * a_ref[...] - dis_ref[...] * gsum
    h = p_ref[:, 0:do] - p_ref[:, 2 * do:3 * do] + y + b_ref[...]
    return jnp.maximum(h, 0.0)


def _b3_b1(g, a, p, b, dis, coef, wc_next, do, do_next):
    def body(g_ref, a_ref, p_ref, b_ref, dis_ref, coef_ref, wc_ref,
             pn_ref, un_ref):
        h = _b3_h(g_ref, a_ref, p_ref, b_ref, dis_ref, coef_ref, do)
        pn = jnp.dot(h, wc_ref[...], preferred_element_type=_f32)
        pn_ref[...] = pn
        un_ref[...] = dis_ref[...] * pn[:, 2 * do_next:3 * do_next]
    outs = (jax.ShapeDtypeStruct((_NP, 3 * do_next), _f32),
            jax.ShapeDtypeStruct((_NP, do_next), _f32))
    return _tc(body, outs, "b3_b1")(g, a, p, b, dis, coef, wc_next)


def _b3_a1(g, a, p, b, dis, coef, do):
    def body(g_ref, a_ref, p_ref, b_ref, dis_ref, coef_ref, h_ref, u_ref):
        h = _b3_h(g_ref, a_ref, p_ref, b_ref, dis_ref, coef_ref, do)
        h_ref[...] = h
        u_ref[...] = dis_ref[...] * h
    outs = (jax.ShapeDtypeStruct((_NP, do), _f32),
            jax.ShapeDtypeStruct((_NP, do), _f32))
    return _tc(body, outs, "b3_a1")(g, a, p, b, dis, coef)


def _a2(g, x, dis, coef, di):
    def body(g_ref, x_ref, dis_ref, coef_ref, t1_ref, u1_ref):
        dis_v = dis_ref[...]
        gsum = g_ref[0] + g_ref[1]
        t1 = coef_ref[...] * x_ref[...] - dis_v * gsum
        t1_ref[...] = t1
        u1_ref[...] = dis_v * t1
    outs = (jax.ShapeDtypeStruct((_NP, di), _f32),
            jax.ShapeDtypeStruct((_NP, di), _f32))
    return _tc(body, outs, "a2")(g, x, dis, coef)


def _a3_h(g1_ref, t1_ref, x_ref, w_ref, b_ref, dis_ref, coef_ref):
    g1 = g1_ref[0] + g1_ref[1]
    t1 = t1_ref[...]
    xv = x_ref[...]
    t2 = 2.0 * (coef_ref[...] * t1 - dis_ref[...] * g1) - xv
    h = (jnp.dot(xv, w_ref[0], preferred_element_type=_f32)
         + jnp.dot(t1, w_ref[1], preferred_element_type=_f32)
         + jnp.dot(t2, w_ref[2], preferred_element_type=_f32)
         + b_ref[...])
    return jnp.maximum(h, 0.0)


def _a3_b1(g1, t1, x, w, b, dis, coef, wc_next, do, do_next):
    def body(g1_ref, t1_ref, x_ref, w_ref, b_ref, dis_ref, coef_ref,
             wc_ref, pn_ref, un_ref):
        h = _a3_h(g1_ref, t1_ref, x_ref, w_ref, b_ref, dis_ref, coef_ref)
        pn = jnp.dot(h, wc_ref[...], preferred_element_type=_f32)
        pn_ref[...] = pn
        un_ref[...] = dis_ref[...] * pn[:, 2 * do_next:3 * do_next]
    outs = (jax.ShapeDtypeStruct((_NP, 3 * do_next), _f32),
            jax.ShapeDtypeStruct((_NP, do_next), _f32))
    return _tc(body, outs, "a3_b1")(g1, t1, x, w, b, dis, coef, wc_next)


def _a3_final(g1, t1, x, w, b, dis, coef, lwt, lb, bat):
    def body(g1_ref, t1_ref, x_ref, w_ref, b_ref, dis_ref, coef_ref,
             lwt_ref, lb_ref, bat_ref, out_ref):
        h = _a3_h(g1_ref, t1_ref, x_ref, w_ref, b_ref, dis_ref, coef_ref)
        seg = lax.broadcasted_iota(jnp.int32, (16, _NP), 0)
        oh = (seg == bat_ref[...]).astype(_f32)
        sums = jnp.dot(oh, h, preferred_element_type=_f32)
        cnts = jnp.sum(oh, axis=1, keepdims=True)
        pooled = sums / jnp.maximum(cnts, 1.0)
        logits = jnp.dot(pooled, lwt_ref[...],
                         preferred_element_type=_f32) + lb_ref[...]
        m = jnp.max(logits, axis=1, keepdims=True)
        e = jnp.exp(logits - m)
        out_ref[...] = logits - m - jnp.log(jnp.sum(e, axis=1, keepdims=True))
    return _tc(body, jax.ShapeDtypeStruct((16, 10), _f32),
               "a3_final")(g1, t1, x, w, b, dis, coef, lwt, lb, bat)


def kernel(x, W1, b1, W2, b2, W3, b3, W4, b4, W5, b5, linW, linb,
           edge_index, batch):
    x = x.astype(_f32)
    row = edge_index[0].astype(jnp.int32)
    col = edge_index[1].astype(jnp.int32)
    npad = _EPAD - _E
    fill = jnp.full((npad,), _N, jnp.int32)
    rowf = jnp.concatenate([row, fill]).reshape(_EPAD // _CH, _CH)
    colf = jnp.concatenate([col, fill]).reshape(_EPAD // _CH, _CH)
    selff = _selfidx(rowf, colf)
    rowp = rowf.reshape(_NW, _NCH, _CH)
    colp = colf.reshape(_NW, _NCH, _CH)
    selfp = selff.reshape(_NW, _NCH, _CH)

    ones_ch = jnp.ones((_CH, 16), _f32)
    zeros = {wi: jnp.zeros((_NP, wi), _f32) for wi in (16, 32, 64)}

    dparts = _sc_scatter(16, False)(ones_ch, zeros[16], rowp)
    sparts = _sc_scatter(16, False)(ones_ch, zeros[16], selfp)

    def g_of(u, wi):
        return _sc_scatter(wi, True)(u, zeros[wi], rowp, colp)

    xp = jnp.pad(x, ((0, _NP - _N), (0, 0)))
    wc1 = jnp.concatenate([W1[0], W1[1], W1[2]], axis=1)
    wc2 = jnp.concatenate([W2[0], W2[1], W2[2]], axis=1)
    wc4 = jnp.concatenate([W4[0], W4[1], W4[2]], axis=1)
    b1r, b2r, b4r = b1.reshape(1, -1), b2.reshape(1, -1), b4.reshape(1, -1)

    dis, coef, p, u = _deg_b1(dparts, sparts, xp, wc1, 32)
    a, ua = _b2(g_of(u, 32), p, dis, coef, 32)
    p, u = _b3_b1(g_of(ua, 32), a, p, b1r, dis, coef, wc2, 32, 32)
    a, ua = _b2(g_of(u, 32), p, dis, coef, 32)
    h, u = _b3_a1(g_of(ua, 32), a, p, b2r, dis, coef, 32)
    t1, u1 = _a2(g_of(u, 32), h, dis, coef, 32)
    p, u = _a3_b1(g_of(u1, 32), t1, h, W3, b3.reshape(1, -1), dis, coef,
                  wc4, 64, 64)
    a, ua = _b2(g_of(u, 64), p, dis, coef, 64)
    h, u = _b3_a1(g_of(ua, 64), a, p, b4r, dis, coef, 64)
    t1, u1 = _a2(g_of(u, 64), h, dis, coef, 64)
    batp = jnp.concatenate(
        [batch.astype(jnp.int32), jnp.full((_NP - _N,), 16, jnp.int32)]
    ).reshape(1, _NP)
    return _a3_final(g_of(u1, 64), t1, h, W5, b5.reshape(1, -1), dis, coef,
                     linW.T, linb.reshape(1, -1), batp)


# speedup vs baseline: 8.5308x; 1.0031x over previous
import functools

import jax
import jax.numpy as jnp
from jax import lax
from jax.experimental import pallas as pl
from jax.experimental.pallas import tpu as pltpu
from jax.experimental.pallas import tpu_sc as plsc

_N = 10000
_NP = 10240
_E = 320000
_EPAD = 327680
_NC, _NS = 2, 16
_NW = _NC * _NS
_CH = 128
_NCH = _EPAD // (_NW * _CH)
_RPT = _NP // _NS

_f32 = jnp.float32


@functools.cache
def _sc_scatter(w):
    mesh = plsc.VectorSubcoreMesh(core_axis_name="c", subcore_axis_name="s")
    nbuf = 8
    la = nbuf // 2
    scratch = [
        pltpu.VMEM((_NCH, _CH), jnp.int32),
        pltpu.VMEM((_NCH, _CH), jnp.int32),
        pltpu.VMEM((nbuf, _CH, w), _f32),
        pltpu.VMEM_SHARED((_NP, w), _f32),
    ] + [pltpu.SemaphoreType.DMA] * (2 * nbuf)

    def body(u_hbm, z_hbm, row_hbm, col_hbm, out_hbm,
             rowi, coli, gbuf, acc, *sems):
        ssem, gsem = sems[:nbuf], sems[nbuf:]
        cid = lax.axis_index("c")
        sid = lax.axis_index("s")
        wid = cid * _NS + sid
        pltpu.sync_copy(col_hbm.at[wid], coli)
        pltpu.sync_copy(row_hbm.at[wid], rowi)
        pltpu.sync_copy(z_hbm.at[pl.ds(sid * _RPT, _RPT)],
                        acc.at[pl.ds(sid * _RPT, _RPT)])
        plsc.subcore_barrier()

        def gstart(j, b):
            pltpu.async_copy(u_hbm.at[rowi.at[j]], gbuf.at[b], gsem[b])

        def gwait(j, b):
            pltpu.make_async_copy(u_hbm.at[rowi.at[j]], gbuf.at[b],
                                  gsem[b]).wait()

        def sstart(j, b):
            pltpu.async_copy(gbuf.at[b], acc.at[coli.at[j]], ssem[b],
                             add=True)

        def swait(j, b):
            pltpu.make_async_copy(gbuf.at[b], acc.at[coli.at[j]],
                                  ssem[b]).wait()

        for j in range(la):
            gstart(j, j)
        for j in range(la):
            gstart(j + la, (j + la) % nbuf)
            gwait(j, j % nbuf)
            sstart(j, j % nbuf)

        @pl.loop(la, _NCH - la, step=nbuf)
        def _(g):
            for b in range(nbuf):
                j = g + b
                bcur = (la + b) % nbuf
                bplus = (la + b + la) % nbuf
                swait(j + la - nbuf, bplus)
                gstart(j + la, bplus)
                gwait(j, bcur)
                sstart(j, bcur)

        for j in range(_NCH - la, _NCH):
            swait(j + la - nbuf, (j + la) % nbuf)
            gwait(j, j % nbuf)
            sstart(j, j % nbuf)
        for j in range(_NCH - la, _NCH):
            swait(j, j % nbuf)
        plsc.subcore_barrier()
        pltpu.sync_copy(acc.at[pl.ds(sid * _RPT, _RPT)],
                        out_hbm.at[cid, pl.ds(sid * _RPT, _RPT)])

    return pl.kernel(
        body,
        out_type=jax.ShapeDtypeStruct((_NC, _NP, w), _f32),
        mesh=mesh,
        scratch_types=scratch,
        compiler_params=pltpu.CompilerParams(use_tc_tiling_on_sc=False),
    )


@functools.cache
def _sc_degree():
    mesh = plsc.VectorSubcoreMesh(core_axis_name="c", subcore_axis_name="s")
    scratch = [
        pltpu.VMEM((_NCH, _CH), jnp.int32),
        pltpu.VMEM((_NCH, _CH), jnp.int32),
        pltpu.VMEM((_NP,), _f32),
        pltpu.VMEM((_NP,), _f32),
    ]

    def body(row_hbm, col_hbm, out_hbm, rowi, coli, dinc, scnt):
        cid = lax.axis_index("c")
        sid = lax.axis_index("s")
        wid = cid * _NS + sid
        pltpu.sync_copy(row_hbm.at[wid], rowi)
        pltpu.sync_copy(col_hbm.at[wid], coli)
        zero = jnp.zeros((16,), _f32)

        @pl.loop(0, _NP // 16)
        def _(i):
            dinc[pl.ds(i * 16, 16)] = zero
            scnt[pl.ds(i * 16, 16)] = zero

        ones = jnp.ones((16,), _f32)

        @pl.loop(0, _NCH)
        def _(j):
            for k in range(_CH // 16):
                r = rowi[j, pl.ds(16 * k, 16)]
                c = coli[j, pl.ds(16 * k, 16)]
                plsc.addupdate_scatter(dinc, [r], ones)
                plsc.addupdate_scatter(scnt, [r], ones, mask=r == c)

        pltpu.sync_copy(dinc, out_hbm.at[wid, 0])
        pltpu.sync_copy(scnt, out_hbm.at[wid, 1])

    return pl.kernel(
        body,
        out_type=jax.ShapeDtypeStruct((_NW, 2, _NP), _f32),
        mesh=mesh,
        scratch_types=scratch,
        compiler_params=pltpu.CompilerParams(use_tc_tiling_on_sc=False,
                                             needs_layout_passes=False),
    )


def _tc(body, out_shapes, name):
    return pl.pallas_call(body, out_shape=out_shapes, name=name)


def _deg_combine(dparts):
    def body(dp, o_ref):
        o_ref[...] = jnp.sum(dp[...], axis=0)
    return _tc(body, jax.ShapeDtypeStruct((2, _NP), _f32),
               "deg_combine")(dparts)


def _deg_b1(dsum, x, wc, do):
    def body(dp, x_ref, wc_ref, dis_ref, coef_ref, p_ref, u_ref):
        dincl = dp[0]
        scnt = dp[1]
        deg = dincl - scnt
        pos = deg > 0
        dis = jnp.where(pos, 1.0 / jnp.sqrt(jnp.maximum(deg, 1e-12)), 0.0)
        dis_ref[...] = dis
        coef_ref[...] = jnp.where(pos, scnt * dis * dis, -1.0)
        p = jnp.dot(x_ref[...], wc_ref[...], preferred_element_type=_f32)
        p_ref[...] = p
        u_ref[...] = dis * p[:, 2 * do:3 * do]
    outs = (jax.ShapeDtypeStruct((_NP, 1), _f32),
            jax.ShapeDtypeStruct((_NP, 1), _f32),
            jax.ShapeDtypeStruct((_NP, 3 * do), _f32),
            jax.ShapeDtypeStruct((_NP, do), _f32))
    return _tc(body, outs, "deg_b1")(dsum, x, wc)


def _b2(g, p, dis, coef, do):
    def body(g_ref, p_ref, dis_ref, coef_ref, a_ref, ua_ref):
        dis_v = dis_ref[...]
        gsum = g_ref[0] + g_ref[1]
        p2 = p_ref[:, 2 * do:3 * do]
        t = coef_ref[...] * p2 - dis_v * gsum
        a = p_ref[:, do:2 * do] + 2.0 * t
        a_ref[...] = a
        ua_ref[...] = dis_v * a
    outs = (jax.ShapeDtypeStruct((_NP, do), _f32),
            jax.ShapeDtypeStruct((_NP, do), _f32))
    return _tc(body, outs, "b2")(g, p, dis, coef)


def _b3_h(g_ref, a_ref, p_ref, b_ref, dis_ref, coef_ref, do):
    gsum = g_ref[0] + g_ref[1]
    y = coef_ref[...] * a_ref[...] - dis_ref[...] * gsum
    h = p_ref[:, 0:do] - p_ref[:, 2 * do:3 * do] + y + b_ref[...]
    return jnp.maximum(h, 0.0)


def _b3_b1(g, a, p, b, dis, coef, wc_next, do, do_next):
    def body(g_ref, a_ref, p_ref, b_ref, dis_ref, coef_ref, wc_ref,
             pn_ref, un_ref):
        h = _b3_h(g_ref, a_ref, p_ref, b_ref, dis_ref, coef_ref, do)
        pn = jnp.dot(h, wc_ref[...], preferred_element_type=_f32)
        pn_ref[...] = pn
        un_ref[...] = dis_ref[...] * pn[:, 2 * do_next:3 * do_next]
    outs = (jax.ShapeDtypeStruct((_NP, 3 * do_next), _f32),
            jax.ShapeDtypeStruct((_NP, do_next), _f32))
    return _tc(body, outs, "b3_b1")(g, a, p, b, dis, coef, wc_next)


def _b3_a1(g, a, p, b, dis, coef, do):
    def body(g_ref, a_ref, p_ref, b_ref, dis_ref, coef_ref, h_ref, u_ref):
        h = _b3_h(g_ref, a_ref, p_ref, b_ref, dis_ref, coef_ref, do)
        h_ref[...] = h
        u_ref[...] = dis_ref[...] * h
    outs = (jax.ShapeDtypeStruct((_NP, do), _f32),
            jax.ShapeDtypeStruct((_NP, do), _f32))
    return _tc(body, outs, "b3_a1")(g, a, p, b, dis, coef)


def _a2(g, x, dis, coef, di):
    def body(g_ref, x_ref, dis_ref, coef_ref, t1_ref, u1_ref):
        dis_v = dis_ref[...]
        gsum = g_ref[0] + g_ref[1]
        t1 = coef_ref[...] * x_ref[...] - dis_v * gsum
        t1_ref[...] = t1
        u1_ref[...] = dis_v * t1
    outs = (jax.ShapeDtypeStruct((_NP, di), _f32),
            jax.ShapeDtypeStruct((_NP, di), _f32))
    return _tc(body, outs, "a2")(g, x, dis, coef)


def _a3_h(g1_ref, t1_ref, x_ref, w_ref, b_ref, dis_ref, coef_ref):
    g1 = g1_ref[0] + g1_ref[1]
    t1 = t1_ref[...]
    xv = x_ref[...]
    t2 = 2.0 * (coef_ref[...] * t1 - dis_ref[...] * g1) - xv
    h = (jnp.dot(xv, w_ref[0], preferred_element_type=_f32)
         + jnp.dot(t1, w_ref[1], preferred_element_type=_f32)
         + jnp.dot(t2, w_ref[2], preferred_element_type=_f32)
         + b_ref[...])
    return jnp.maximum(h, 0.0)


def _a3_b1(g1, t1, x, w, b, dis, coef, wc_next, do, do_next):
    def body(g1_ref, t1_ref, x_ref, w_ref, b_ref, dis_ref, coef_ref,
             wc_ref, pn_ref, un_ref):
        h = _a3_h(g1_ref, t1_ref, x_ref, w_ref, b_ref, dis_ref, coef_ref)
        pn = jnp.dot(h, wc_ref[...], preferred_element_type=_f32)
        pn_ref[...] = pn
        un_ref[...] = dis_ref[...] * pn[:, 2 * do_next:3 * do_next]
    outs = (jax.ShapeDtypeStruct((_NP, 3 * do_next), _f32),
            jax.ShapeDtypeStruct((_NP, do_next), _f32))
    return _tc(body, outs, "a3_b1")(g1, t1, x, w, b, dis, coef, wc_next)


def _a3_final(g1, t1, x, w, b, dis, coef, lwt, lb, bat):
    def body(g1_ref, t1_ref, x_ref, w_ref, b_ref, dis_ref, coef_ref,
             lwt_ref, lb_ref, bat_ref, out_ref):
        h = _a3_h(g1_ref, t1_ref, x_ref, w_ref, b_ref, dis_ref, coef_ref)
        seg = lax.broadcasted_iota(jnp.int32, (16, _NP), 0)
        oh = (seg == bat_ref[...]).astype(_f32)
        sums = jnp.dot(oh, h, preferred_element_type=_f32)
        cnts = jnp.sum(oh, axis=1, keepdims=True)
        pooled = sums / jnp.maximum(cnts, 1.0)
        logits = jnp.dot(pooled, lwt_ref[...],
                         preferred_element_type=_f32) + lb_ref[...]
        m = jnp.max(logits, axis=1, keepdims=True)
        e = jnp.exp(logits - m)
        out_ref[...] = logits - m - jnp.log(jnp.sum(e, axis=1, keepdims=True))
    return _tc(body, jax.ShapeDtypeStruct((16, 10), _f32),
               "a3_final")(g1, t1, x, w, b, dis, coef, lwt, lb, bat)


def kernel(x, W1, b1, W2, b2, W3, b3, W4, b4, W5, b5, linW, linb,
           edge_index, batch):
    x = x.astype(_f32)
    row = edge_index[0].astype(jnp.int32)
    col = edge_index[1].astype(jnp.int32)
    npad = _EPAD - _E
    fill = jnp.full((npad,), _N, jnp.int32)
    rowp = jnp.concatenate([row, fill]).reshape(_NW, _NCH, _CH)
    colp = jnp.concatenate([col, fill]).reshape(_NW, _NCH, _CH)

    zeros = {wi: jnp.zeros((_NP, wi), _f32) for wi in (32, 64)}

    dsum = _deg_combine(_sc_degree()(rowp, colp)).reshape(2, _NP, 1)

    def g_of(u, wi):
        return _sc_scatter(wi)(u, zeros[wi], rowp, colp)

    xp = jnp.pad(x, ((0, _NP - _N), (0, 0)))
    wc1 = jnp.concatenate([W1[0], W1[1], W1[2]], axis=1)
    wc2 = jnp.concatenate([W2[0], W2[1], W2[2]], axis=1)
    wc4 = jnp.concatenate([W4[0], W4[1], W4[2]], axis=1)
    b1r, b2r, b4r = b1.reshape(1, -1), b2.reshape(1, -1), b4.reshape(1, -1)

    dis, coef, p, u = _deg_b1(dsum, xp, wc1, 32)
    a, ua = _b2(g_of(u, 32), p, dis, coef, 32)
    p, u = _b3_b1(g_of(ua, 32), a, p, b1r, dis, coef, wc2, 32, 32)
    a, ua = _b2(g_of(u, 32), p, dis, coef, 32)
    h, u = _b3_a1(g_of(ua, 32), a, p, b2r, dis, coef, 32)
    t1, u1 = _a2(g_of(u, 32), h, dis, coef, 32)
    p, u = _a3_b1(g_of(u1, 32), t1, h, W3, b3.reshape(1, -1), dis, coef,
                  wc4, 64, 64)
    a, ua = _b2(g_of(u, 64), p, dis, coef, 64)
    h, u = _b3_a1(g_of(ua, 64), a, p, b4r, dis, coef, 64)
    t1, u1 = _a2(g_of(u, 64), h, dis, coef, 64)
    batp = jnp.concatenate(
        [batch.astype(jnp.int32), jnp.full((_NP - _N,), 16, jnp.int32)]
    ).reshape(1, _NP)
    return _a3_final(g_of(u1, 64), t1, h, W5, b5.reshape(1, -1), dis, coef,
                     linW.T, linb.reshape(1, -1), batp)


# speedup vs baseline: 8.5768x; 1.0054x over previous
import functools

import jax
import jax.numpy as jnp
from jax import lax
from jax.experimental import pallas as pl
from jax.experimental.pallas import tpu as pltpu
from jax.experimental.pallas import tpu_sc as plsc

_N = 10000
_NP = 10240
_E = 320000
_EPAD = 327680
_NC, _NS = 2, 16
_NW = _NC * _NS
_CH = 128
_NCH = _EPAD // (_NW * _CH)
_RPT = _NP // _NS

_f32 = jnp.float32


@functools.cache
def _sc_scatter(w):
    mesh = plsc.VectorSubcoreMesh(core_axis_name="c", subcore_axis_name="s")
    nbuf = 4
    la = nbuf // 2
    scratch = [
        pltpu.VMEM((_NCH, _CH), jnp.int32),
        pltpu.VMEM((_NCH, _CH), jnp.int32),
        pltpu.VMEM((nbuf, _CH, w), _f32),
        pltpu.VMEM_SHARED((_NP, w), _f32),
    ] + [pltpu.SemaphoreType.DMA] * (2 * nbuf)

    def body(u_hbm, z_hbm, row_hbm, col_hbm, out_hbm,
             rowi, coli, gbuf, acc, *sems):
        ssem, gsem = sems[:nbuf], sems[nbuf:]
        cid = lax.axis_index("c")
        sid = lax.axis_index("s")
        wid = cid * _NS + sid
        pltpu.sync_copy(col_hbm.at[wid], coli)
        pltpu.sync_copy(row_hbm.at[wid], rowi)
        pltpu.sync_copy(z_hbm.at[pl.ds(sid * _RPT, _RPT)],
                        acc.at[pl.ds(sid * _RPT, _RPT)])
        plsc.subcore_barrier()

        def gstart(j, b):
            pltpu.async_copy(u_hbm.at[rowi.at[j]], gbuf.at[b], gsem[b])

        def gwait(j, b):
            pltpu.make_async_copy(u_hbm.at[rowi.at[j]], gbuf.at[b],
                                  gsem[b]).wait()

        def sstart(j, b):
            pltpu.async_copy(gbuf.at[b], acc.at[coli.at[j]], ssem[b],
                             add=True)

        def swait(j, b):
            pltpu.make_async_copy(gbuf.at[b], acc.at[coli.at[j]],
                                  ssem[b]).wait()

        for j in range(la):
            gstart(j, j)
        for j in range(la):
            gstart(j + la, (j + la) % nbuf)
            gwait(j, j % nbuf)
            sstart(j, j % nbuf)

        @pl.loop(la, _NCH - la, step=nbuf)
        def _(g):
            for b in range(nbuf):
                j = g + b
                bcur = (la + b) % nbuf
                bplus = (la + b + la) % nbuf
                swait(j + la - nbuf, bplus)
                gstart(j + la, bplus)
                gwait(j, bcur)
                sstart(j, bcur)

        for j in range(_NCH - la, _NCH):
            swait(j + la - nbuf, (j + la) % nbuf)
            gwait(j, j % nbuf)
            sstart(j, j % nbuf)
        for j in range(_NCH - la, _NCH):
            swait(j, j % nbuf)
        plsc.subcore_barrier()
        pltpu.sync_copy(acc.at[pl.ds(sid * _RPT, _RPT)],
                        out_hbm.at[cid, pl.ds(sid * _RPT, _RPT)])

    return pl.kernel(
        body,
        out_type=jax.ShapeDtypeStruct((_NC, _NP, w), _f32),
        mesh=mesh,
        scratch_types=scratch,
        compiler_params=pltpu.CompilerParams(use_tc_tiling_on_sc=False),
    )


@functools.cache
def _sc_degree():
    mesh = plsc.VectorSubcoreMesh(core_axis_name="c", subcore_axis_name="s")
    scratch = [
        pltpu.VMEM((_NCH, _CH), jnp.int32),
        pltpu.VMEM((_NCH, _CH), jnp.int32),
        pltpu.VMEM((_NP,), _f32),
        pltpu.VMEM((_NP,), _f32),
    ]

    def body(row_hbm, col_hbm, out_hbm, rowi, coli, dinc, scnt):
        cid = lax.axis_index("c")
        sid = lax.axis_index("s")
        wid = cid * _NS + sid
        pltpu.sync_copy(row_hbm.at[wid], rowi)
        pltpu.sync_copy(col_hbm.at[wid], coli)
        zero = jnp.zeros((16,), _f32)

        @pl.loop(0, _NP // 16)
        def _(i):
            dinc[pl.ds(i * 16, 16)] = zero
            scnt[pl.ds(i * 16, 16)] = zero

        ones = jnp.ones((16,), _f32)

        @pl.loop(0, _NCH)
        def _(j):
            for k in range(_CH // 16):
                r = rowi[j, pl.ds(16 * k, 16)]
                c = coli[j, pl.ds(16 * k, 16)]
                plsc.addupdate_scatter(dinc, [r], ones)
                plsc.addupdate_scatter(scnt, [r], ones, mask=r == c)

        pltpu.sync_copy(dinc, out_hbm.at[wid, 0])
        pltpu.sync_copy(scnt, out_hbm.at[wid, 1])

    return pl.kernel(
        body,
        out_type=jax.ShapeDtypeStruct((_NW, 2, _NP), _f32),
        mesh=mesh,
        scratch_types=scratch,
        compiler_params=pltpu.CompilerParams(use_tc_tiling_on_sc=False,
                                             needs_layout_passes=False),
    )


def _tc(body, out_shapes, name):
    return pl.pallas_call(body, out_shape=out_shapes, name=name)


def _deg_combine(dparts):
    def body(dp, o_ref):
        o_ref[...] = jnp.sum(dp[...], axis=0)
    return _tc(body, jax.ShapeDtypeStruct((2, _NP), _f32),
               "deg_combine")(dparts)


def _deg_b1(dsum, x, wc, do):
    def body(dp, x_ref, wc_ref, dis_ref, coef_ref, p_ref, u_ref):
        dincl = dp[0]
        scnt = dp[1]
        deg = dincl - scnt
        pos = deg > 0
        dis = jnp.where(pos, 1.0 / jnp.sqrt(jnp.maximum(deg, 1e-12)), 0.0)
        dis_ref[...] = dis
        coef_ref[...] = jnp.where(pos, scnt * dis * dis, -1.0)
        p = jnp.dot(x_ref[...], wc_ref[...], preferred_element_type=_f32)
        p_ref[...] = p
        u_ref[...] = dis * p[:, 2 * do:3 * do]
    outs = (jax.ShapeDtypeStruct((_NP, 1), _f32),
            jax.ShapeDtypeStruct((_NP, 1), _f32),
            jax.ShapeDtypeStruct((_NP, 3 * do), _f32),
            jax.ShapeDtypeStruct((_NP, do), _f32))
    return _tc(body, outs, "deg_b1")(dsum, x, wc)


def _b2(g, p, dis, coef, do):
    def body(g_ref, p_ref, dis_ref, coef_ref, a_ref, ua_ref):
        dis_v = dis_ref[...]
        gsum = g_ref[0] + g_ref[1]
        p2 = p_ref[:, 2 * do:3 * do]
        t = coef_ref[...] * p2 - dis_v * gsum
        a = p_ref[:, do:2 * do] + 2.0 * t
        a_ref[...] = a
        ua_ref[...] = dis_v * a
    outs = (jax.ShapeDtypeStruct((_NP, do), _f32),
            jax.ShapeDtypeStruct((_NP, do), _f32))
    return _tc(body, outs, "b2")(g, p, dis, coef)


def _b3_h(g_ref, a_ref, p_ref, b_ref, dis_ref, coef_ref, do):
    gsum = g_ref[0] + g_ref[1]
    y = coef_ref[...] * a_ref[...] - dis_ref[...] * gsum
    h = p_ref[:, 0:do] - p_ref[:, 2 * do:3 * do] + y + b_ref[...]
    return jnp.maximum(h, 0.0)


def _b3_b1(g, a, p, b, dis, coef, wc_next, do, do_next):
    def body(g_ref, a_ref, p_ref, b_ref, dis_ref, coef_ref, wc_ref,
             pn_ref, un_ref):
        h = _b3_h(g_ref, a_ref, p_ref, b_ref, dis_ref, coef_ref, do)
        pn = jnp.dot(h, wc_ref[...], preferred_element_type=_f32)
        pn_ref[...] = pn
        un_ref[...] = dis_ref[...] * pn[:, 2 * do_next:3 * do_next]
    outs = (jax.ShapeDtypeStruct((_NP, 3 * do_next), _f32),
            jax.ShapeDtypeStruct((_NP, do_next), _f32))
    return _tc(body, outs, "b3_b1")(g, a, p, b, dis, coef, wc_next)


def _b3_a1(g, a, p, b, dis, coef, do):
    def body(g_ref, a_ref, p_ref, b_ref, dis_ref, coef_ref, h_ref, u_ref):
        h = _b3_h(g_ref, a_ref, p_ref, b_ref, dis_ref, coef_ref, do)
        h_ref[...] = h
        u_ref[...] = dis_ref[...] * h
    outs = (jax.ShapeDtypeStruct((_NP, do), _f32),
            jax.ShapeDtypeStruct((_NP, do), _f32))
    return _tc(body, outs, "b3_a1")(g, a, p, b, dis, coef)


def _a2(g, x, dis, coef, di):
    def body(g_ref, x_ref, dis_ref, coef_ref, t1_ref, u1_ref):
        dis_v = dis_ref[...]
        gsum = g_ref[0] + g_ref[1]
        t1 = coef_ref[...] * x_ref[...] - dis_v * gsum
        t1_ref[...] = t1
        u1_ref[...] = dis_v * t1
    outs = (jax.ShapeDtypeStruct((_NP, di), _f32),
            jax.ShapeDtypeStruct((_NP, di), _f32))
    return _tc(body, outs, "a2")(g, x, dis, coef)


def _a3_h(g1_ref, t1_ref, x_ref, w_ref, b_ref, dis_ref, coef_ref):
    g1 = g1_ref[0] + g1_ref[1]
    t1 = t1_ref[...]
    xv = x_ref[...]
    t2 = 2.0 * (coef_ref[...] * t1 - dis_ref[...] * g1) - xv
    h = (jnp.dot(xv, w_ref[0], preferred_element_type=_f32)
         + jnp.dot(t1, w_ref[1], preferred_element_type=_f32)
         + jnp.dot(t2, w_ref[2], preferred_element_type=_f32)
         + b_ref[...])
    return jnp.maximum(h, 0.0)


def _a3_b1(g1, t1, x, w, b, dis, coef, wc_next, do, do_next):
    def body(g1_ref, t1_ref, x_ref, w_ref, b_ref, dis_ref, coef_ref,
             wc_ref, pn_ref, un_ref):
        h = _a3_h(g1_ref, t1_ref, x_ref, w_ref, b_ref, dis_ref, coef_ref)
        pn = jnp.dot(h, wc_ref[...], preferred_element_type=_f32)
        pn_ref[...] = pn
        un_ref[...] = dis_ref[...] * pn[:, 2 * do_next:3 * do_next]
    outs = (jax.ShapeDtypeStruct((_NP, 3 * do_next), _f32),
            jax.ShapeDtypeStruct((_NP, do_next), _f32))
    return _tc(body, outs, "a3_b1")(g1, t1, x, w, b, dis, coef, wc_next)


def _a3_final(g1, t1, x, w, b, dis, coef, lwt, lb, bat):
    def body(g1_ref, t1_ref, x_ref, w_ref, b_ref, dis_ref, coef_ref,
             lwt_ref, lb_ref, bat_ref, out_ref):
        h = _a3_h(g1_ref, t1_ref, x_ref, w_ref, b_ref, dis_ref, coef_ref)
        seg = lax.broadcasted_iota(jnp.int32, (16, _NP), 0)
        oh = (seg == bat_ref[...]).astype(_f32)
        sums = jnp.dot(oh, h, preferred_element_type=_f32)
        cnts = jnp.sum(oh, axis=1, keepdims=True)
        pooled = sums / jnp.maximum(cnts, 1.0)
        logits = jnp.dot(pooled, lwt_ref[...],
                         preferred_element_type=_f32) + lb_ref[...]
        m = jnp.max(logits, axis=1, keepdims=True)
        e = jnp.exp(logits - m)
        out_ref[...] = logits - m - jnp.log(jnp.sum(e, axis=1, keepdims=True))
    return _tc(body, jax.ShapeDtypeStruct((16, 10), _f32),
               "a3_final")(g1, t1, x, w, b, dis, coef, lwt, lb, bat)


def kernel(x, W1, b1, W2, b2, W3, b3, W4, b4, W5, b5, linW, linb,
           edge_index, batch):
    x = x.astype(_f32)
    row = edge_index[0].astype(jnp.int32)
    col = edge_index[1].astype(jnp.int32)
    npad = _EPAD - _E
    fill = jnp.full((npad,), _N, jnp.int32)
    rowp = jnp.concatenate([row, fill]).reshape(_NW, _NCH, _CH)
    colp = jnp.concatenate([col, fill]).reshape(_NW, _NCH, _CH)

    zeros = {wi: jnp.zeros((_NP, wi), _f32) for wi in (32, 64)}

    dsum = _deg_combine(_sc_degree()(rowp, colp)).reshape(2, _NP, 1)

    def g_of(u, wi):
        return _sc_scatter(wi)(u, zeros[wi], rowp, colp)

    xp = jnp.pad(x, ((0, _NP - _N), (0, 0)))
    wc1 = jnp.concatenate([W1[0], W1[1], W1[2]], axis=1)
    wc2 = jnp.concatenate([W2[0], W2[1], W2[2]], axis=1)
    wc4 = jnp.concatenate([W4[0], W4[1], W4[2]], axis=1)
    b1r, b2r, b4r = b1.reshape(1, -1), b2.reshape(1, -1), b4.reshape(1, -1)

    dis, coef, p, u = _deg_b1(dsum, xp, wc1, 32)
    a, ua = _b2(g_of(u, 32), p, dis, coef, 32)
    p, u = _b3_b1(g_of(ua, 32), a, p, b1r, dis, coef, wc2, 32, 32)
    a, ua = _b2(g_of(u, 32), p, dis, coef, 32)
    h, u = _b3_a1(g_of(ua, 32), a, p, b2r, dis, coef, 32)
    t1, u1 = _a2(g_of(u, 32), h, dis, coef, 32)
    p, u = _a3_b1(g_of(u1, 32), t1, h, W3, b3.reshape(1, -1), dis, coef,
                  wc4, 64, 64)
    a, ua = _b2(g_of(u, 64), p, dis, coef, 64)
    h, u = _b3_a1(g_of(ua, 64), a, p, b4r, dis, coef, 64)
    t1, u1 = _a2(g_of(u, 64), h, dis, coef, 64)
    batp = jnp.concatenate(
        [batch.astype(jnp.int32), jnp.full((_NP - _N,), 16, jnp.int32)]
    ).reshape(1, _NP)
    return _a3_final(g_of(u1, 64), t1, h, W5, b5.reshape(1, -1), dis, coef,
                     linW.T, linb.reshape(1, -1), batp)


# speedup vs baseline: 12.2843x; 1.4323x over previous
import functools

import jax
import jax.numpy as jnp
from jax import lax
from jax.experimental import pallas as pl
from jax.experimental.pallas import tpu as pltpu
from jax.experimental.pallas import tpu_sc as plsc

_N = 10000
_NP = 10240
_E = 320000
_EPAD = 327680
_NC, _NS = 2, 16
_NW = _NC * _NS
_CH = 128
_NCH = _EPAD // (_NW * _CH)
_RPT = _NP // _NS

_f32 = jnp.float32


@functools.cache
def _sc_scatter(w, stage=False):
    mesh = plsc.VectorSubcoreMesh(core_axis_name="c", subcore_axis_name="s")
    nbuf = 4
    la = nbuf // 2
    scratch = [
        pltpu.VMEM((_NCH, _CH), jnp.int32),
        pltpu.VMEM((_NCH, _CH), jnp.int32),
        pltpu.VMEM((nbuf, _CH, w), _f32),
        pltpu.VMEM_SHARED((_NP, w if stage else 1), _f32),
        pltpu.VMEM_SHARED((_NP, w), _f32),
    ] + [pltpu.SemaphoreType.DMA] * (2 * nbuf)

    def body(u_hbm, z_hbm, row_hbm, col_hbm, out_hbm,
             rowi, coli, gbuf, u_sp, acc, *sems):
        ssem, gsem = sems[:nbuf], sems[nbuf:]
        cid = lax.axis_index("c")
        sid = lax.axis_index("s")
        wid = cid * _NS + sid
        pltpu.sync_copy(col_hbm.at[wid], coli)
        pltpu.sync_copy(row_hbm.at[wid], rowi)
        if stage:
            pltpu.sync_copy(u_hbm.at[pl.ds(sid * _RPT, _RPT)],
                            u_sp.at[pl.ds(sid * _RPT, _RPT)])
        pltpu.sync_copy(z_hbm.at[pl.ds(sid * _RPT, _RPT)],
                        acc.at[pl.ds(sid * _RPT, _RPT)])
        plsc.subcore_barrier()

        u_src = u_sp if stage else u_hbm

        def gstart(j, b):
            pltpu.async_copy(u_src.at[rowi.at[j]], gbuf.at[b], gsem[b])

        def gwait(j, b):
            pltpu.make_async_copy(u_src.at[rowi.at[j]], gbuf.at[b],
                                  gsem[b]).wait()

        def sstart(j, b):
            pltpu.async_copy(gbuf.at[b], acc.at[coli.at[j]], ssem[b],
                             add=True)

        def swait(j, b):
            pltpu.make_async_copy(gbuf.at[b], acc.at[coli.at[j]],
                                  ssem[b]).wait()

        for j in range(la):
            gstart(j, j)
        for j in range(la):
            gstart(j + la, (j + la) % nbuf)
            gwait(j, j % nbuf)
            sstart(j, j % nbuf)

        @pl.loop(la, _NCH - la, step=nbuf)
        def _(g):
            for b in range(nbuf):
                j = g + b
                bcur = (la + b) % nbuf
                bplus = (la + b + la) % nbuf
                swait(j + la - nbuf, bplus)
                gstart(j + la, bplus)
                gwait(j, bcur)
                sstart(j, bcur)

        for j in range(_NCH - la, _NCH):
            swait(j + la - nbuf, (j + la) % nbuf)
            gwait(j, j % nbuf)
            sstart(j, j % nbuf)
        for j in range(_NCH - la, _NCH):
            swait(j, j % nbuf)
        plsc.subcore_barrier()
        pltpu.sync_copy(acc.at[pl.ds(sid * _RPT, _RPT)],
                        out_hbm.at[cid, pl.ds(sid * _RPT, _RPT)])

    return pl.kernel(
        body,
        out_type=jax.ShapeDtypeStruct((_NC, _NP, w), _f32),
        mesh=mesh,
        scratch_types=scratch,
        compiler_params=pltpu.CompilerParams(use_tc_tiling_on_sc=False),
    )


@functools.cache
def _sc_degree():
    mesh = plsc.VectorSubcoreMesh(core_axis_name="c", subcore_axis_name="s")
    scratch = [
        pltpu.VMEM((_NCH, _CH), jnp.int32),
        pltpu.VMEM((_NCH, _CH), jnp.int32),
        pltpu.VMEM((_NP,), _f32),
        pltpu.VMEM((_NP,), _f32),
    ]

    def body(row_hbm, col_hbm, out_hbm, rowi, coli, dinc, scnt):
        cid = lax.axis_index("c")
        sid = lax.axis_index("s")
        wid = cid * _NS + sid
        pltpu.sync_copy(row_hbm.at[wid], rowi)
        pltpu.sync_copy(col_hbm.at[wid], coli)
        zero = jnp.zeros((16,), _f32)

        @pl.loop(0, _NP // 16)
        def _(i):
            dinc[pl.ds(i * 16, 16)] = zero
            scnt[pl.ds(i * 16, 16)] = zero

        ones = jnp.ones((16,), _f32)

        @pl.loop(0, _NCH)
        def _(j):
            for k in range(_CH // 16):
                r = rowi[j, pl.ds(16 * k, 16)]
                c = coli[j, pl.ds(16 * k, 16)]
                plsc.addupdate_scatter(dinc, [r], ones)
                plsc.addupdate_scatter(scnt, [r], ones, mask=r == c)

        pltpu.sync_copy(dinc, out_hbm.at[wid, 0])
        pltpu.sync_copy(scnt, out_hbm.at[wid, 1])

    return pl.kernel(
        body,
        out_type=jax.ShapeDtypeStruct((_NW, 2, _NP), _f32),
        mesh=mesh,
        scratch_types=scratch,
        compiler_params=pltpu.CompilerParams(use_tc_tiling_on_sc=False,
                                             needs_layout_passes=False),
    )


def _tc(body, out_shapes, name):
    return pl.pallas_call(body, out_shape=out_shapes, name=name)


def _deg_combine(dparts):
    def body(dp, o_ref):
        o_ref[...] = jnp.sum(dp[...], axis=0)
    return _tc(body, jax.ShapeDtypeStruct((2, _NP), _f32),
               "deg_combine")(dparts)


def _deg_b1(dsum, x, wc, do):
    def body(dp, x_ref, wc_ref, dis_ref, coef_ref, p_ref, u_ref):
        dincl = dp[0]
        scnt = dp[1]
        deg = dincl - scnt
        pos = deg > 0
        dis = jnp.where(pos, 1.0 / jnp.sqrt(jnp.maximum(deg, 1e-12)), 0.0)
        dis_ref[...] = dis
        coef_ref[...] = jnp.where(pos, scnt * dis * dis, -1.0)
        p = jnp.dot(x_ref[...], wc_ref[...], preferred_element_type=_f32)
        p_ref[...] = p
        u_ref[...] = dis * p[:, 2 * do:3 * do]
    outs = (jax.ShapeDtypeStruct((_NP, 1), _f32),
            jax.ShapeDtypeStruct((_NP, 1), _f32),
            jax.ShapeDtypeStruct((_NP, 3 * do), _f32),
            jax.ShapeDtypeStruct((_NP, do), _f32))
    return _tc(body, outs, "deg_b1")(dsum, x, wc)


def _b2(g, p, dis, coef, do):
    def body(g_ref, p_ref, dis_ref, coef_ref, a_ref, ua_ref):
        dis_v = dis_ref[...]
        gsum = g_ref[0] + g_ref[1]
        p2 = p_ref[:, 2 * do:3 * do]
        t = coef_ref[...] * p2 - dis_v * gsum
        a = p_ref[:, do:2 * do] + 2.0 * t
        a_ref[...] = a
        ua_ref[...] = dis_v * a
    outs = (jax.ShapeDtypeStruct((_NP, do), _f32),
            jax.ShapeDtypeStruct((_NP, do), _f32))
    return _tc(body, outs, "b2")(g, p, dis, coef)


def _b3_h(g_ref, a_ref, p_ref, b_ref, dis_ref, coef_ref, do):
    gsum = g_ref[0] + g_ref[1]
    y = coef_ref[...] * a_ref[...] - dis_ref[...] * gsum
    h = p_ref[:, 0:do] - p_ref[:, 2 * do:3 * do] + y + b_ref[...]
    return jnp.maximum(h, 0.0)


def _b3_b1(g, a, p, b, dis, coef, wc_next, do, do_next):
    def body(g_ref, a_ref, p_ref, b_ref, dis_ref, coef_ref, wc_ref,
             pn_ref, un_ref):
        h = _b3_h(g_ref, a_ref, p_ref, b_ref, dis_ref, coef_ref, do)
        pn = jnp.dot(h, wc_ref[...], preferred_element_type=_f32)
        pn_ref[...] = pn
        un_ref[...] = dis_ref[...] * pn[:, 2 * do_next:3 * do_next]
    outs = (jax.ShapeDtypeStruct((_NP, 3 * do_next), _f32),
            jax.ShapeDtypeStruct((_NP, do_next), _f32))
    return _tc(body, outs, "b3_b1")(g, a, p, b, dis, coef, wc_next)


def _b3_a1(g, a, p, b, dis, coef, do):
    def body(g_ref, a_ref, p_ref, b_ref, dis_ref, coef_ref, h_ref, u_ref):
        h = _b3_h(g_ref, a_ref, p_ref, b_ref, dis_ref, coef_ref, do)
        h_ref[...] = h
        u_ref[...] = dis_ref[...] * h
    outs = (jax.ShapeDtypeStruct((_NP, do), _f32),
            jax.ShapeDtypeStruct((_NP, do), _f32))
    return _tc(body, outs, "b3_a1")(g, a, p, b, dis, coef)


def _a2(g, x, dis, coef, di):
    def body(g_ref, x_ref, dis_ref, coef_ref, t1_ref, u1_ref):
        dis_v = dis_ref[...]
        gsum = g_ref[0] + g_ref[1]
        t1 = coef_ref[...] * x_ref[...] - dis_v * gsum
        t1_ref[...] = t1
        u1_ref[...] = dis_v * t1
    outs = (jax.ShapeDtypeStruct((_NP, di), _f32),
            jax.ShapeDtypeStruct((_NP, di), _f32))
    return _tc(body, outs, "a2")(g, x, dis, coef)


def _a3_h(g1_ref, t1_ref, x_ref, w_ref, b_ref, dis_ref, coef_ref):
    g1 = g1_ref[0] + g1_ref[1]
    t1 = t1_ref[...]
    xv = x_ref[...]
    t2 = 2.0 * (coef_ref[...] * t1 - dis_ref[...] * g1) - xv
    h = (jnp.dot(xv, w_ref[0], preferred_element_type=_f32)
         + jnp.dot(t1, w_ref[1], preferred_element_type=_f32)
         + jnp.dot(t2, w_ref[2], preferred_element_type=_f32)
         + b_ref[...])
    return jnp.maximum(h, 0.0)


def _a3_b1(g1, t1, x, w, b, dis, coef, wc_next, do, do_next):
    def body(g1_ref, t1_ref, x_ref, w_ref, b_ref, dis_ref, coef_ref,
             wc_ref, pn_ref, un_ref):
        h = _a3_h(g1_ref, t1_ref, x_ref, w_ref, b_ref, dis_ref, coef_ref)
        pn = jnp.dot(h, wc_ref[...], preferred_element_type=_f32)
        pn_ref[...] = pn
        un_ref[...] = dis_ref[...] * pn[:, 2 * do_next:3 * do_next]
    outs = (jax.ShapeDtypeStruct((_NP, 3 * do_next), _f32),
            jax.ShapeDtypeStruct((_NP, do_next), _f32))
    return _tc(body, outs, "a3_b1")(g1, t1, x, w, b, dis, coef, wc_next)


def _a3_final(g1, t1, x, w, b, dis, coef, lwt, lb, bat):
    def body(g1_ref, t1_ref, x_ref, w_ref, b_ref, dis_ref, coef_ref,
             lwt_ref, lb_ref, bat_ref, out_ref):
        h = _a3_h(g1_ref, t1_ref, x_ref, w_ref, b_ref, dis_ref, coef_ref)
        seg = lax.broadcasted_iota(jnp.int32, (16, _NP), 0)
        oh = (seg == bat_ref[...]).astype(_f32)
        sums = jnp.dot(oh, h, preferred_element_type=_f32)
        cnts = jnp.sum(oh, axis=1, keepdims=True)
        pooled = sums / jnp.maximum(cnts, 1.0)
        logits = jnp.dot(pooled, lwt_ref[...],
                         preferred_element_type=_f32) + lb_ref[...]
        m = jnp.max(logits, axis=1, keepdims=True)
        e = jnp.exp(logits - m)
        out_ref[...] = logits - m - jnp.log(jnp.sum(e, axis=1, keepdims=True))
    return _tc(body, jax.ShapeDtypeStruct((16, 10), _f32),
               "a3_final")(g1, t1, x, w, b, dis, coef, lwt, lb, bat)


def kernel(x, W1, b1, W2, b2, W3, b3, W4, b4, W5, b5, linW, linb,
           edge_index, batch):
    x = x.astype(_f32)
    row = edge_index[0].astype(jnp.int32)
    col = edge_index[1].astype(jnp.int32)
    npad = _EPAD - _E
    fill = jnp.full((npad,), _N, jnp.int32)
    rowp = jnp.concatenate([row, fill]).reshape(_NW, _NCH, _CH)
    colp = jnp.concatenate([col, fill]).reshape(_NW, _NCH, _CH)

    zeros = {wi: jnp.zeros((_NP, wi), _f32) for wi in (32, 64)}

    dsum = _deg_combine(_sc_degree()(rowp, colp)).reshape(2, _NP, 1)

    def g_of(u, wi):
        return _sc_scatter(wi, stage=(wi <= 32))(u, zeros[wi], rowp, colp)

    xp = jnp.pad(x, ((0, _NP - _N), (0, 0)))
    wc1 = jnp.concatenate([W1[0], W1[1], W1[2]], axis=1)
    wc2 = jnp.concatenate([W2[0], W2[1], W2[2]], axis=1)
    wc4 = jnp.concatenate([W4[0], W4[1], W4[2]], axis=1)
    b1r, b2r, b4r = b1.reshape(1, -1), b2.reshape(1, -1), b4.reshape(1, -1)

    dis, coef, p, u = _deg_b1(dsum, xp, wc1, 32)
    a, ua = _b2(g_of(u, 32), p, dis, coef, 32)
    p, u = _b3_b1(g_of(ua, 32), a, p, b1r, dis, coef, wc2, 32, 32)
    a, ua = _b2(g_of(u, 32), p, dis, coef, 32)
    h, u = _b3_a1(g_of(ua, 32), a, p, b2r, dis, coef, 32)
    t1, u1 = _a2(g_of(u, 32), h, dis, coef, 32)
    p, u = _a3_b1(g_of(u1, 32), t1, h, W3, b3.reshape(1, -1), dis, coef,
                  wc4, 64, 64)
    a, ua = _b2(g_of(u, 64), p, dis, coef, 64)
    h, u = _b3_a1(g_of(ua, 64), a, p, b4r, dis, coef, 64)
    t1, u1 = _a2(g_of(u, 64), h, dis, coef, 64)
    batp = jnp.concatenate(
        [batch.astype(jnp.int32), jnp.full((_NP - _N,), 16, jnp.int32)]
    ).reshape(1, _NP)
    return _a3_final(g_of(u1, 64), t1, h, W5, b5.reshape(1, -1), dis, coef,
                     linW.T, linb.reshape(1, -1), batp)


# speedup vs baseline: 17.6019x; 1.4329x over previous
import functools

import jax
import jax.numpy as jnp
from jax import lax
from jax.experimental import pallas as pl
from jax.experimental.pallas import tpu as pltpu
from jax.experimental.pallas import tpu_sc as plsc

_N = 10000
_NP = 10240
_E = 320000
_EPAD = 327680
_NC, _NS = 2, 16
_NW = _NC * _NS
_CH = 128
_NCH = _EPAD // (_NW * _CH)
_RPT = _NP // _NS

_f32 = jnp.float32


@functools.cache
def _sc_scatter(w, stage=False):
    mesh = plsc.VectorSubcoreMesh(core_axis_name="c", subcore_axis_name="s")
    nbuf = 4
    la = nbuf // 2
    scratch = [
        pltpu.VMEM((_NCH, _CH), jnp.int32),
        pltpu.VMEM((_NCH, _CH), jnp.int32),
        pltpu.VMEM((nbuf, _CH, w), _f32),
        pltpu.VMEM_SHARED((_NP, w if stage else 1), _f32),
        pltpu.VMEM_SHARED((_NP, w), _f32),
    ] + [pltpu.SemaphoreType.DMA] * (2 * nbuf)

    def body(u_hbm, z_hbm, row_hbm, col_hbm, out_hbm,
             rowi, coli, gbuf, u_sp, acc, *sems):
        ssem, gsem = sems[:nbuf], sems[nbuf:]
        cid = lax.axis_index("c")
        sid = lax.axis_index("s")
        wid = cid * _NS + sid
        pltpu.sync_copy(col_hbm.at[wid], coli)
        pltpu.sync_copy(row_hbm.at[wid], rowi)
        if stage:
            pltpu.sync_copy(u_hbm.at[pl.ds(sid * _RPT, _RPT)],
                            u_sp.at[pl.ds(sid * _RPT, _RPT)])
        pltpu.sync_copy(z_hbm.at[pl.ds(sid * _RPT, _RPT)],
                        acc.at[pl.ds(sid * _RPT, _RPT)])
        plsc.subcore_barrier()

        u_src = u_sp if stage else u_hbm

        def gstart(j, b):
            pltpu.async_copy(u_src.at[rowi.at[j]], gbuf.at[b], gsem[b])

        def gwait(j, b):
            pltpu.make_async_copy(u_src.at[rowi.at[j]], gbuf.at[b],
                                  gsem[b]).wait()

        def sstart(j, b):
            pltpu.async_copy(gbuf.at[b], acc.at[coli.at[j]], ssem[b],
                             add=True)

        def swait(j, b):
            pltpu.make_async_copy(gbuf.at[b], acc.at[coli.at[j]],
                                  ssem[b]).wait()

        for j in range(la):
            gstart(j, j)
        for j in range(la):
            gstart(j + la, (j + la) % nbuf)
            gwait(j, j % nbuf)
            sstart(j, j % nbuf)

        @pl.loop(la, _NCH - la, step=nbuf)
        def _(g):
            for b in range(nbuf):
                j = g + b
                bcur = (la + b) % nbuf
                bplus = (la + b + la) % nbuf
                swait(j + la - nbuf, bplus)
                gstart(j + la, bplus)
                gwait(j, bcur)
                sstart(j, bcur)

        for j in range(_NCH - la, _NCH):
            swait(j + la - nbuf, (j + la) % nbuf)
            gwait(j, j % nbuf)
            sstart(j, j % nbuf)
        for j in range(_NCH - la, _NCH):
            swait(j, j % nbuf)
        plsc.subcore_barrier()
        pltpu.sync_copy(acc.at[pl.ds(sid * _RPT, _RPT)],
                        out_hbm.at[cid, pl.ds(sid * _RPT, _RPT)])

    return pl.kernel(
        body,
        out_type=jax.ShapeDtypeStruct((_NC, _NP, w), _f32),
        mesh=mesh,
        scratch_types=scratch,
        compiler_params=pltpu.CompilerParams(use_tc_tiling_on_sc=False),
    )


@functools.cache
def _sc_degree():
    mesh = plsc.VectorSubcoreMesh(core_axis_name="c", subcore_axis_name="s")
    scratch = [
        pltpu.VMEM((_NCH, _CH), jnp.int32),
        pltpu.VMEM((_NCH, _CH), jnp.int32),
        pltpu.VMEM((_NP,), _f32),
        pltpu.VMEM((_NP,), _f32),
    ]

    def body(row_hbm, col_hbm, out_hbm, rowi, coli, dinc, scnt):
        cid = lax.axis_index("c")
        sid = lax.axis_index("s")
        wid = cid * _NS + sid
        pltpu.sync_copy(row_hbm.at[wid], rowi)
        pltpu.sync_copy(col_hbm.at[wid], coli)
        zero = jnp.zeros((16,), _f32)

        @pl.loop(0, _NP // 16)
        def _(i):
            dinc[pl.ds(i * 16, 16)] = zero
            scnt[pl.ds(i * 16, 16)] = zero

        ones = jnp.ones((16,), _f32)

        @pl.loop(0, _NCH)
        def _(j):
            for k in range(_CH // 16):
                r = rowi[j, pl.ds(16 * k, 16)]
                c = coli[j, pl.ds(16 * k, 16)]
                plsc.addupdate_scatter(dinc, [r], ones)
                plsc.addupdate_scatter(scnt, [r], ones, mask=r == c)

        pltpu.sync_copy(dinc, out_hbm.at[wid, 0])
        pltpu.sync_copy(scnt, out_hbm.at[wid, 1])

    return pl.kernel(
        body,
        out_type=jax.ShapeDtypeStruct((_NW, 2, _NP), _f32),
        mesh=mesh,
        scratch_types=scratch,
        compiler_params=pltpu.CompilerParams(use_tc_tiling_on_sc=False,
                                             needs_layout_passes=False),
    )


def _tc(body, out_shapes, name):
    return pl.pallas_call(body, out_shape=out_shapes, name=name)


def _deg_combine(dparts):
    def body(dp, o_ref):
        o_ref[...] = jnp.sum(dp[...], axis=0)
    return _tc(body, jax.ShapeDtypeStruct((2, _NP), _f32),
               "deg_combine")(dparts)


def _deg_b1(dsum, x, wc, do):
    def body(dp, x_ref, wc_ref, dis_ref, coef_ref, p_ref, u_ref):
        dincl = dp[0]
        scnt = dp[1]
        deg = dincl - scnt
        pos = deg > 0
        dis = jnp.where(pos, 1.0 / jnp.sqrt(jnp.maximum(deg, 1e-12)), 0.0)
        dis_ref[...] = dis
        coef_ref[...] = jnp.where(pos, scnt * dis * dis, -1.0)
        p = jnp.dot(x_ref[...], wc_ref[...], preferred_element_type=_f32)
        p_ref[...] = p
        u_ref[...] = dis * p[:, 2 * do:3 * do]
    outs = (jax.ShapeDtypeStruct((_NP, 1), _f32),
            jax.ShapeDtypeStruct((_NP, 1), _f32),
            jax.ShapeDtypeStruct((_NP, 3 * do), _f32),
            jax.ShapeDtypeStruct((_NP, do), _f32))
    return _tc(body, outs, "deg_b1")(dsum, x, wc)


def _b2(g, p, dis, coef, do):
    def body(g_ref, p_ref, dis_ref, coef_ref, a_ref, ua_ref):
        dis_v = dis_ref[...]
        gsum = g_ref[0] + g_ref[1]
        p2 = p_ref[:, 2 * do:3 * do]
        t = coef_ref[...] * p2 - dis_v * gsum
        a = p_ref[:, do:2 * do] + 2.0 * t
        a_ref[...] = a
        ua_ref[...] = dis_v * a
    outs = (jax.ShapeDtypeStruct((_NP, do), _f32),
            jax.ShapeDtypeStruct((_NP, do), _f32))
    return _tc(body, outs, "b2")(g, p, dis, coef)


def _b3_h(g_ref, a_ref, p_ref, b_ref, dis_ref, coef_ref, do):
    gsum = g_ref[0] + g_ref[1]
    y = coef_ref[...] * a_ref[...] - dis_ref[...] * gsum
    h = p_ref[:, 0:do] - p_ref[:, 2 * do:3 * do] + y + b_ref[...]
    return jnp.maximum(h, 0.0)


def _b3_b1(g, a, p, b, dis, coef, wc_next, do, do_next):
    def body(g_ref, a_ref, p_ref, b_ref, dis_ref, coef_ref, wc_ref,
             pn_ref, un_ref):
        h = _b3_h(g_ref, a_ref, p_ref, b_ref, dis_ref, coef_ref, do)
        pn = jnp.dot(h, wc_ref[...], preferred_element_type=_f32)
        pn_ref[...] = pn
        un_ref[...] = dis_ref[...] * pn[:, 2 * do_next:3 * do_next]
    outs = (jax.ShapeDtypeStruct((_NP, 3 * do_next), _f32),
            jax.ShapeDtypeStruct((_NP, do_next), _f32))
    return _tc(body, outs, "b3_b1")(g, a, p, b, dis, coef, wc_next)


def _b3_a1(g, a, p, b, dis, coef, do):
    def body(g_ref, a_ref, p_ref, b_ref, dis_ref, coef_ref, h_ref, u_ref):
        h = _b3_h(g_ref, a_ref, p_ref, b_ref, dis_ref, coef_ref, do)
        h_ref[...] = h
        u_ref[...] = dis_ref[...] * h
    outs = (jax.ShapeDtypeStruct((_NP, do), _f32),
            jax.ShapeDtypeStruct((_NP, do), _f32))
    return _tc(body, outs, "b3_a1")(g, a, p, b, dis, coef)


def _a2(g, x, dis, coef, di):
    def body(g_ref, x_ref, dis_ref, coef_ref, t1_ref, u1_ref):
        dis_v = dis_ref[...]
        gsum = g_ref[0] + g_ref[1]
        t1 = coef_ref[...] * x_ref[...] - dis_v * gsum
        t1_ref[...] = t1
        u1_ref[...] = dis_v * t1
    outs = (jax.ShapeDtypeStruct((_NP, di), _f32),
            jax.ShapeDtypeStruct((_NP, di), _f32))
    return _tc(body, outs, "a2")(g, x, dis, coef)


def _a3_h(g1_ref, t1_ref, x_ref, w_ref, b_ref, dis_ref, coef_ref):
    g1 = g1_ref[0] + g1_ref[1]
    t1 = t1_ref[...]
    xv = x_ref[...]
    t2 = 2.0 * (coef_ref[...] * t1 - dis_ref[...] * g1) - xv
    h = (jnp.dot(xv, w_ref[0], preferred_element_type=_f32)
         + jnp.dot(t1, w_ref[1], preferred_element_type=_f32)
         + jnp.dot(t2, w_ref[2], preferred_element_type=_f32)
         + b_ref[...])
    return jnp.maximum(h, 0.0)


def _a3_b1(g1, t1, x, w, b, dis, coef, wc_next, do, do_next):
    def body(g1_ref, t1_ref, x_ref, w_ref, b_ref, dis_ref, coef_ref,
             wc_ref, pn_ref, un_ref):
        h = _a3_h(g1_ref, t1_ref, x_ref, w_ref, b_ref, dis_ref, coef_ref)
        pn = jnp.dot(h, wc_ref[...], preferred_element_type=_f32)
        pn_ref[...] = pn
        un_ref[...] = dis_ref[...] * pn[:, 2 * do_next:3 * do_next]
    outs = (jax.ShapeDtypeStruct((_NP, 3 * do_next), _f32),
            jax.ShapeDtypeStruct((_NP, do_next), _f32))
    return _tc(body, outs, "a3_b1")(g1, t1, x, w, b, dis, coef, wc_next)


def _a3_final(g1, t1, x, w, b, dis, coef, lwt, lb, bat):
    def body(g1_ref, t1_ref, x_ref, w_ref, b_ref, dis_ref, coef_ref,
             lwt_ref, lb_ref, bat_ref, out_ref):
        h = _a3_h(g1_ref, t1_ref, x_ref, w_ref, b_ref, dis_ref, coef_ref)
        seg = lax.broadcasted_iota(jnp.int32, (16, _NP), 0)
        oh = (seg == bat_ref[...]).astype(_f32)
        sums = jnp.dot(oh, h, preferred_element_type=_f32)
        cnts = jnp.sum(oh, axis=1, keepdims=True)
        pooled = sums / jnp.maximum(cnts, 1.0)
        logits = jnp.dot(pooled, lwt_ref[...],
                         preferred_element_type=_f32) + lb_ref[...]
        m = jnp.max(logits, axis=1, keepdims=True)
        e = jnp.exp(logits - m)
        out_ref[...] = logits - m - jnp.log(jnp.sum(e, axis=1, keepdims=True))
    return _tc(body, jax.ShapeDtypeStruct((16, 10), _f32),
               "a3_final")(g1, t1, x, w, b, dis, coef, lwt, lb, bat)


def kernel(x, W1, b1, W2, b2, W3, b3, W4, b4, W5, b5, linW, linb,
           edge_index, batch):
    x = x.astype(_f32)
    row = edge_index[0].astype(jnp.int32)
    col = edge_index[1].astype(jnp.int32)
    npad = _EPAD - _E
    fill = jnp.full((npad,), _N, jnp.int32)
    rowp = jnp.concatenate([row, fill]).reshape(_NW, _NCH, _CH)
    colp = jnp.concatenate([col, fill]).reshape(_NW, _NCH, _CH)

    zeros = {wi: jnp.zeros((_NP, wi), _f32) for wi in (32,)}

    dsum = _deg_combine(_sc_degree()(rowp, colp)).reshape(2, _NP, 1)

    def g_of(u, wi):
        if wi > 32:
            halves = [g_of(u[:, k:k + 32], 32) for k in range(0, wi, 32)]
            return jnp.concatenate(halves, axis=2)
        return _sc_scatter(wi, stage=True)(u, zeros[wi], rowp, colp)

    xp = jnp.pad(x, ((0, _NP - _N), (0, 0)))
    wc1 = jnp.concatenate([W1[0], W1[1], W1[2]], axis=1)
    wc2 = jnp.concatenate([W2[0], W2[1], W2[2]], axis=1)
    wc4 = jnp.concatenate([W4[0], W4[1], W4[2]], axis=1)
    b1r, b2r, b4r = b1.reshape(1, -1), b2.reshape(1, -1), b4.reshape(1, -1)

    dis, coef, p, u = _deg_b1(dsum, xp, wc1, 32)
    a, ua = _b2(g_of(u, 32), p, dis, coef, 32)
    p, u = _b3_b1(g_of(ua, 32), a, p, b1r, dis, coef, wc2, 32, 32)
    a, ua = _b2(g_of(u, 32), p, dis, coef, 32)
    h, u = _b3_a1(g_of(ua, 32), a, p, b2r, dis, coef, 32)
    t1, u1 = _a2(g_of(u, 32), h, dis, coef, 32)
    p, u = _a3_b1(g_of(u1, 32), t1, h, W3, b3.reshape(1, -1), dis, coef,
                  wc4, 64, 64)
    a, ua = _b2(g_of(u, 64), p, dis, coef, 64)
    h, u = _b3_a1(g_of(ua, 64), a, p, b4r, dis, coef, 64)
    t1, u1 = _a2(g_of(u, 64), h, dis, coef, 64)
    batp = jnp.concatenate(
        [batch.astype(jnp.int32), jnp.full((_NP - _N,), 16, jnp.int32)]
    ).reshape(1, _NP)
    return _a3_final(g_of(u1, 64), t1, h, W5, b5.reshape(1, -1), dis, coef,
                     linW.T, linb.reshape(1, -1), batp)
